# initial kernel scaffold (unmeasured)
import jax
import jax.numpy as jnp
from jax import lax
from jax.experimental import pallas as pl
from jax.experimental.pallas import tpu as pltpu

N_DEV = 8


def kernel(x, w_mat):
    m_tot, k_per = x.shape
    _, n = w_mat.shape
    m_per = m_tot // N_DEV

    def body(x_ref, w_ref, out_ref, comm_ref, amax_src, amax_buf,
             send_sems, recv_sems, amax_send_sems, amax_recv_sems,
             credit_sem):
        p = lax.axis_index("i")
        left = lax.rem(p + N_DEV - 1, N_DEV)
        right = lax.rem(p + 1, N_DEV)

        barrier_sem = pltpu.get_barrier_semaphore()
        for nbr in (left, right):
            pl.semaphore_signal(
                barrier_sem, inc=1,
                device_id=(nbr,), device_id_type=pl.DeviceIdType.MESH,
            )
        pl.semaphore_wait(barrier_sem, 2)

        amax_buf[...] = jnp.zeros_like(amax_buf)

        def gemm_chunk(c):
            xc = x_ref[pl.ds(c * m_per, m_per), :]
            return jnp.dot(
                xc, w_ref[...],
                preferred_element_type=jnp.float32,
                precision=lax.Precision.HIGHEST,
            )

        for s in range(N_DEV - 1):
            send_slot = s % 2
            recv_slot = (s + 1) % 2
            c = lax.rem(p - (s + 1) + 2 * N_DEV, N_DEV)
            g = gemm_chunk(c)
            if s == 0:
                comm_ref[send_slot] = g
            else:
                comm_ref[send_slot] = comm_ref[send_slot] + g
            if s >= 1:
                pl.semaphore_wait(credit_sem, 1)
            rdma = pltpu.make_async_remote_copy(
                src_ref=comm_ref.at[send_slot],
                dst_ref=comm_ref.at[recv_slot],
                send_sem=send_sems.at[send_slot],
                recv_sem=recv_sems.at[recv_slot],
                device_id=(right,),
                device_id_type=pl.DeviceIdType.MESH,
            )
            rdma.start()
            rdma.wait()
            if s < N_DEV - 2:
                pl.semaphore_signal(
                    credit_sem, inc=1,
                    device_id=(left,), device_id_type=pl.DeviceIdType.MESH,
                )

        y = comm_ref[(N_DEV - 1) % 2] + gemm_chunk(p)

        local_amax = jnp.max(jnp.abs(y))
        amax_src[...] = jnp.full(amax_src.shape, local_amax, jnp.float32)
        out_rdmas = []
        for j in range(1, N_DEV):
            k = lax.rem(p + j, N_DEV)
            r = pltpu.make_async_remote_copy(
                src_ref=amax_src,
                dst_ref=amax_buf.at[p],
                send_sem=amax_send_sems.at[j],
                recv_sem=amax_recv_sems.at[p],
                device_id=(k,),
                device_id_type=pl.DeviceIdType.MESH,
            )
            r.start()
            out_rdmas.append(r)
        for j in range(1, N_DEV):
            k = lax.rem(p + j, N_DEV)
            pltpu.make_async_remote_copy(
                src_ref=amax_src,
                dst_ref=amax_buf.at[k],
                send_sem=amax_send_sems.at[0],
                recv_sem=amax_recv_sems.at[k],
                device_id=(k,),
                device_id_type=pl.DeviceIdType.MESH,
            ).wait_recv()
        for r in out_rdmas:
            r.wait_send()

        global_amax = jnp.maximum(jnp.max(amax_buf[...]), local_amax)
        scale = global_amax / 448.0
        q = (y / scale).astype(jnp.float8_e4m3fn).astype(jnp.float32)
        out_ref[...] = q * scale

    return pl.pallas_call(
        body,
        out_shape=jax.ShapeDtypeStruct((m_per, n), jnp.float32),
        in_specs=[
            pl.BlockSpec(memory_space=pltpu.VMEM),
            pl.BlockSpec(memory_space=pltpu.VMEM),
        ],
        out_specs=pl.BlockSpec(memory_space=pltpu.VMEM),
        scratch_shapes=[
            pltpu.VMEM((2, m_per, n), jnp.float32),
            pltpu.VMEM((8, 128), jnp.float32),
            pltpu.VMEM((N_DEV, 8, 128), jnp.float32),
            pltpu.SemaphoreType.DMA((2,)),
            pltpu.SemaphoreType.DMA((2,)),
            pltpu.SemaphoreType.DMA((N_DEV,)),
            pltpu.SemaphoreType.DMA((N_DEV,)),
            pltpu.SemaphoreType.REGULAR,
        ],
        compiler_params=pltpu.CompilerParams(collective_id=0),
    )(x, w_mat)


# baseline (device time: 402936 ns/iter reference)
import jax
import jax.numpy as jnp
from jax import lax
from jax.experimental import pallas as pl
from jax.experimental.pallas import tpu as pltpu

N_DEV = 8


def kernel(x, w_mat):
    m_tot, k_per = x.shape
    _, n = w_mat.shape
    m_per = m_tot // N_DEV

    def body(x_ref, w_ref, out_ref, comm_ref, amax_src, amax_buf,
             send_sems, recv_sems, amax_send_sems, amax_recv_sems,
             credit_sem):
        p = lax.axis_index("i")
        left = lax.rem(p + N_DEV - 1, N_DEV)
        right = lax.rem(p + 1, N_DEV)

        barrier_sem = pltpu.get_barrier_semaphore()
        for nbr in (left, right):
            pl.semaphore_signal(
                barrier_sem, inc=1,
                device_id=(nbr,), device_id_type=pl.DeviceIdType.MESH,
            )
        pl.semaphore_wait(barrier_sem, 2)

        amax_buf[...] = jnp.zeros(amax_buf.shape, amax_buf.dtype)

        def gemm_chunk(c):
            xc = x_ref[pl.ds(c * m_per, m_per), :]
            return jnp.dot(
                xc, w_ref[...],
                preferred_element_type=jnp.float32,
                precision=lax.Precision.HIGHEST,
            )

        for s in range(N_DEV - 1):
            send_slot = s % 2
            recv_slot = (s + 1) % 2
            c = lax.rem(p - (s + 1) + 2 * N_DEV, N_DEV)
            g = gemm_chunk(c)
            if s == 0:
                comm_ref[send_slot] = g
            else:
                comm_ref[send_slot] = comm_ref[send_slot] + g
            if s >= 1:
                pl.semaphore_wait(credit_sem, 1)
            rdma = pltpu.make_async_remote_copy(
                src_ref=comm_ref.at[send_slot],
                dst_ref=comm_ref.at[recv_slot],
                send_sem=send_sems.at[send_slot],
                recv_sem=recv_sems.at[recv_slot],
                device_id=(right,),
                device_id_type=pl.DeviceIdType.MESH,
            )
            rdma.start()
            rdma.wait()
            if s < N_DEV - 2:
                pl.semaphore_signal(
                    credit_sem, inc=1,
                    device_id=(left,), device_id_type=pl.DeviceIdType.MESH,
                )

        y = comm_ref[(N_DEV - 1) % 2] + gemm_chunk(p)

        local_amax = jnp.max(jnp.abs(y))
        amax_src[...] = jnp.full(amax_src.shape, local_amax, jnp.float32)
        out_rdmas = []
        for j in range(1, N_DEV):
            k = lax.rem(p + j, N_DEV)
            r = pltpu.make_async_remote_copy(
                src_ref=amax_src,
                dst_ref=amax_buf.at[p],
                send_sem=amax_send_sems.at[j],
                recv_sem=amax_recv_sems.at[p],
                device_id=(k,),
                device_id_type=pl.DeviceIdType.MESH,
            )
            r.start()
            out_rdmas.append(r)
        for j in range(1, N_DEV):
            k = lax.rem(p + j, N_DEV)
            pltpu.make_async_remote_copy(
                src_ref=amax_src,
                dst_ref=amax_buf.at[k],
                send_sem=amax_send_sems.at[0],
                recv_sem=amax_recv_sems.at[k],
                device_id=(k,),
                device_id_type=pl.DeviceIdType.MESH,
            ).wait_recv()
        for r in out_rdmas:
            r.wait_send()

        global_amax = jnp.maximum(jnp.max(amax_buf[...]), local_amax)
        scale = global_amax / 448.0
        q = (y / scale).astype(jnp.float8_e4m3fn).astype(jnp.float32)
        out_ref[...] = q * scale

    return pl.pallas_call(
        body,
        out_shape=jax.ShapeDtypeStruct((m_per, n), jnp.float32),
        in_specs=[
            pl.BlockSpec(memory_space=pltpu.VMEM),
            pl.BlockSpec(memory_space=pltpu.VMEM),
        ],
        out_specs=pl.BlockSpec(memory_space=pltpu.VMEM),
        scratch_shapes=[
            pltpu.VMEM((2, m_per, n), jnp.float32),
            pltpu.VMEM((8, 128), jnp.float32),
            pltpu.VMEM((N_DEV, 8, 128), jnp.float32),
            pltpu.SemaphoreType.DMA((2,)),
            pltpu.SemaphoreType.DMA((2,)),
            pltpu.SemaphoreType.DMA((N_DEV,)),
            pltpu.SemaphoreType.DMA((N_DEV,)),
            pltpu.SemaphoreType.REGULAR,
        ],
        compiler_params=pltpu.CompilerParams(collective_id=0),
    )(x, w_mat)


# device time: 197316 ns/iter; 2.0421x vs baseline; 2.0421x over previous
import jax
import jax.numpy as jnp
from jax import lax
from jax.experimental import pallas as pl
from jax.experimental.pallas import tpu as pltpu

N_DEV = 8


def kernel(x, w_mat):
    m_tot, k_per = x.shape
    _, n = w_mat.shape
    m_per = m_tot // N_DEV
    nh = n // 2

    def body(x_ref, w_ref, out_ref, comm_cw, comm_ccw, amax_src, amax_buf,
             cw_send_sems, cw_recv_sems, ccw_send_sems, ccw_recv_sems,
             amax_send_sems, amax_recv_sems, credit_cw, credit_ccw):
        p = lax.axis_index("i")
        left = lax.rem(p + N_DEV - 1, N_DEV)
        right = lax.rem(p + 1, N_DEV)

        barrier_sem = pltpu.get_barrier_semaphore()
        for nbr in (left, right):
            pl.semaphore_signal(
                barrier_sem, inc=1,
                device_id=(nbr,), device_id_type=pl.DeviceIdType.MESH,
            )
        pl.semaphore_wait(barrier_sem, 2)

        amax_buf[...] = jnp.zeros(amax_buf.shape, amax_buf.dtype)

        def gemm_half(c, half):
            xc = x_ref[pl.ds(c * m_per, m_per), :]
            wc = w_ref[:, pl.ds(half * nh, nh)]
            return jnp.dot(
                xc, wc,
                preferred_element_type=jnp.float32,
                precision=lax.Precision.HIGHEST,
            )

        def chunk_cw(s):
            return lax.rem(p - (s + 1) + 2 * N_DEV, N_DEV)

        def chunk_ccw(s):
            return lax.rem(p + (s + 1), N_DEV)

        g_cw = gemm_half(chunk_cw(0), 0)
        g_ccw = gemm_half(chunk_ccw(0), 1)
        comm_cw[0] = g_cw
        comm_ccw[0] = g_ccw

        for s in range(N_DEV - 1):
            send_slot = s % 2
            recv_slot = (s + 1) % 2
            if s >= 1:
                pl.semaphore_wait(credit_cw, 1)
                pl.semaphore_wait(credit_ccw, 1)
            rdma_cw = pltpu.make_async_remote_copy(
                src_ref=comm_cw.at[send_slot],
                dst_ref=comm_cw.at[recv_slot],
                send_sem=cw_send_sems.at[send_slot],
                recv_sem=cw_recv_sems.at[recv_slot],
                device_id=(right,),
                device_id_type=pl.DeviceIdType.MESH,
            )
            rdma_ccw = pltpu.make_async_remote_copy(
                src_ref=comm_ccw.at[send_slot],
                dst_ref=comm_ccw.at[recv_slot],
                send_sem=ccw_send_sems.at[send_slot],
                recv_sem=ccw_recv_sems.at[recv_slot],
                device_id=(left,),
                device_id_type=pl.DeviceIdType.MESH,
            )
            rdma_cw.start()
            rdma_ccw.start()
            if s < N_DEV - 2:
                g_cw = gemm_half(chunk_cw(s + 1), 0)
                g_ccw = gemm_half(chunk_ccw(s + 1), 1)
            else:
                g_cw = gemm_half(p, 0)
                g_ccw = gemm_half(p, 1)
            rdma_cw.wait_send()
            rdma_ccw.wait_send()
            if s < N_DEV - 2:
                pl.semaphore_signal(
                    credit_cw, inc=1,
                    device_id=(left,), device_id_type=pl.DeviceIdType.MESH,
                )
                pl.semaphore_signal(
                    credit_ccw, inc=1,
                    device_id=(right,), device_id_type=pl.DeviceIdType.MESH,
                )
            rdma_cw.wait_recv()
            rdma_ccw.wait_recv()
            if s < N_DEV - 2:
                comm_cw[recv_slot] = comm_cw[recv_slot] + g_cw
                comm_ccw[recv_slot] = comm_ccw[recv_slot] + g_ccw

        last_slot = (N_DEV - 1) % 2
        y = jnp.concatenate(
            [comm_cw[last_slot] + g_cw, comm_ccw[last_slot] + g_ccw], axis=1
        )

        local_amax = jnp.max(jnp.abs(y))
        amax_src[...] = jnp.full(amax_src.shape, local_amax, jnp.float32)
        out_rdmas = []
        for j in range(1, N_DEV):
            k = lax.rem(p + j, N_DEV)
            r = pltpu.make_async_remote_copy(
                src_ref=amax_src,
                dst_ref=amax_buf.at[p],
                send_sem=amax_send_sems.at[j],
                recv_sem=amax_recv_sems.at[p],
                device_id=(k,),
                device_id_type=pl.DeviceIdType.MESH,
            )
            r.start()
            out_rdmas.append(r)
        for j in range(1, N_DEV):
            k = lax.rem(p + j, N_DEV)
            pltpu.make_async_remote_copy(
                src_ref=amax_src,
                dst_ref=amax_buf.at[k],
                send_sem=amax_send_sems.at[0],
                recv_sem=amax_recv_sems.at[k],
                device_id=(k,),
                device_id_type=pl.DeviceIdType.MESH,
            ).wait_recv()
        for r in out_rdmas:
            r.wait_send()

        global_amax = jnp.maximum(jnp.max(amax_buf[...]), local_amax)
        scale = global_amax / 448.0
        q = (y / scale).astype(jnp.float8_e4m3fn).astype(jnp.float32)
        out_ref[...] = q * scale

    return pl.pallas_call(
        body,
        out_shape=jax.ShapeDtypeStruct((m_per, n), jnp.float32),
        in_specs=[
            pl.BlockSpec(memory_space=pltpu.VMEM),
            pl.BlockSpec(memory_space=pltpu.VMEM),
        ],
        out_specs=pl.BlockSpec(memory_space=pltpu.VMEM),
        scratch_shapes=[
            pltpu.VMEM((2, m_per, nh), jnp.float32),
            pltpu.VMEM((2, m_per, nh), jnp.float32),
            pltpu.VMEM((8, 128), jnp.float32),
            pltpu.VMEM((N_DEV, 8, 128), jnp.float32),
            pltpu.SemaphoreType.DMA((2,)),
            pltpu.SemaphoreType.DMA((2,)),
            pltpu.SemaphoreType.DMA((2,)),
            pltpu.SemaphoreType.DMA((2,)),
            pltpu.SemaphoreType.DMA((N_DEV,)),
            pltpu.SemaphoreType.DMA((N_DEV,)),
            pltpu.SemaphoreType.REGULAR,
            pltpu.SemaphoreType.REGULAR,
        ],
        compiler_params=pltpu.CompilerParams(collective_id=0),
    )(x, w_mat)


# device time: 181494 ns/iter; 2.2201x vs baseline; 1.0872x over previous
import jax
import jax.numpy as jnp
from jax import lax
from jax.experimental import pallas as pl
from jax.experimental.pallas import tpu as pltpu

N_DEV = 8
N_SLOT = 4


def kernel(x, w_mat):
    m_tot, k_per = x.shape
    _, n = w_mat.shape
    m_per = m_tot // N_DEV
    nh = n // 2
    nq = nh // 2

    def body(x_ref, w_ref, out_ref, comm_cw, comm_ccw, amax_src, amax_buf,
             cw_send_sems, cw_recv_sems, ccw_send_sems, ccw_recv_sems,
             amax_send_sems, amax_recv_sems, credit_cw, credit_ccw):
        p = lax.axis_index("i")
        left = lax.rem(p + N_DEV - 1, N_DEV)
        right = lax.rem(p + 1, N_DEV)

        barrier_sem = pltpu.get_barrier_semaphore()
        for nbr in (left, right):
            pl.semaphore_signal(
                barrier_sem, inc=1,
                device_id=(nbr,), device_id_type=pl.DeviceIdType.MESH,
            )

        amax_buf[...] = jnp.zeros(amax_buf.shape, amax_buf.dtype)

        def gemm_half(c, half):
            xc = x_ref[pl.ds(c * m_per, m_per), :]
            wc = w_ref[:, pl.ds(half * nh, nh)]
            return jnp.dot(
                xc, wc,
                preferred_element_type=jnp.float32,
                precision=lax.Precision.HIGHEST,
            )

        def chunk_cw(s):
            return lax.rem(p - (s + 1) + 2 * N_DEV, N_DEV)

        def chunk_ccw(s):
            return lax.rem(p + (s + 1), N_DEV)

        g_cw = gemm_half(chunk_cw(0), 0)
        g_ccw = gemm_half(chunk_ccw(0), 1)
        comm_cw[0] = g_cw
        comm_ccw[0] = g_ccw

        pl.semaphore_wait(barrier_sem, 2)

        def make(comm, send_sems, recv_sems, dev, s, u):
            slot = s % N_SLOT
            nxt = (s + 1) % N_SLOT
            return pltpu.make_async_remote_copy(
                src_ref=comm.at[slot, :, pl.ds(u * nq, nq)],
                dst_ref=comm.at[nxt, :, pl.ds(u * nq, nq)],
                send_sem=send_sems.at[slot, u],
                recv_sem=recv_sems.at[nxt, u],
                device_id=(dev,),
                device_id_type=pl.DeviceIdType.MESH,
            )

        prev = None
        for s in range(N_DEV - 1):
            slot = s % N_SLOT
            nxt = (s + 1) % N_SLOT
            if s >= 1:
                pl.semaphore_wait(credit_cw, 1)
                pl.semaphore_wait(credit_ccw, 1)
            cw0 = make(comm_cw, cw_send_sems, cw_recv_sems, right, s, 0)
            cw1 = make(comm_cw, cw_send_sems, cw_recv_sems, right, s, 1)
            ccw0 = make(comm_ccw, ccw_send_sems, ccw_recv_sems, left, s, 0)
            ccw1 = make(comm_ccw, ccw_send_sems, ccw_recv_sems, left, s, 1)
            cw0.start()
            ccw0.start()
            if s == 0:
                cw1.start()
                ccw1.start()
            else:
                prev["cw1"].wait_recv()
                comm_cw[slot, :, nq:] = comm_cw[slot, :, nq:] + g_cw[:, nq:]
                cw1.start()
                prev["ccw1"].wait_recv()
                comm_ccw[slot, :, nq:] = comm_ccw[slot, :, nq:] + g_ccw[:, nq:]
                ccw1.start()
            if s < N_DEV - 2:
                g_cw = gemm_half(chunk_cw(s + 1), 0)
                g_ccw = gemm_half(chunk_ccw(s + 1), 1)
            else:
                g_cw = gemm_half(p, 0)
                g_ccw = gemm_half(p, 1)
            if s >= 1:
                prev["cw0"].wait_send()
                prev["cw1"].wait_send()
                prev["ccw0"].wait_send()
                prev["ccw1"].wait_send()
            if s < N_DEV - 2:
                pl.semaphore_signal(
                    credit_cw, inc=1,
                    device_id=(left,), device_id_type=pl.DeviceIdType.MESH,
                )
                pl.semaphore_signal(
                    credit_ccw, inc=1,
                    device_id=(right,), device_id_type=pl.DeviceIdType.MESH,
                )
            cw0.wait_recv()
            comm_cw[nxt, :, :nq] = comm_cw[nxt, :, :nq] + g_cw[:, :nq]
            ccw0.wait_recv()
            comm_ccw[nxt, :, :nq] = comm_ccw[nxt, :, :nq] + g_ccw[:, :nq]
            prev = {"cw0": cw0, "cw1": cw1, "ccw0": ccw0, "ccw1": ccw1}

        last = (N_DEV - 1) % N_SLOT
        prev["cw1"].wait_recv()
        comm_cw[last, :, nq:] = comm_cw[last, :, nq:] + g_cw[:, nq:]
        prev["ccw1"].wait_recv()
        comm_ccw[last, :, nq:] = comm_ccw[last, :, nq:] + g_ccw[:, nq:]
        prev["cw0"].wait_send()
        prev["cw1"].wait_send()
        prev["ccw0"].wait_send()
        prev["ccw1"].wait_send()

        y_l = comm_cw[last]
        y_r = comm_ccw[last]

        local_amax = jnp.maximum(jnp.max(jnp.abs(y_l)), jnp.max(jnp.abs(y_r)))
        amax_src[...] = jnp.full(amax_src.shape, local_amax, jnp.float32)
        out_rdmas = []
        for j in range(1, N_DEV):
            k = lax.rem(p + j, N_DEV)
            r = pltpu.make_async_remote_copy(
                src_ref=amax_src,
                dst_ref=amax_buf.at[p],
                send_sem=amax_send_sems.at[j],
                recv_sem=amax_recv_sems.at[p],
                device_id=(k,),
                device_id_type=pl.DeviceIdType.MESH,
            )
            r.start()
            out_rdmas.append(r)
        for j in range(1, N_DEV):
            k = lax.rem(p + j, N_DEV)
            pltpu.make_async_remote_copy(
                src_ref=amax_src,
                dst_ref=amax_buf.at[k],
                send_sem=amax_send_sems.at[0],
                recv_sem=amax_recv_sems.at[k],
                device_id=(k,),
                device_id_type=pl.DeviceIdType.MESH,
            ).wait_recv()
        for r in out_rdmas:
            r.wait_send()

        global_amax = jnp.maximum(jnp.max(amax_buf[...]), local_amax)
        scale = global_amax / 448.0
        out_ref[:, :nh] = (y_l / scale).astype(jnp.float8_e4m3fn).astype(
            jnp.float32) * scale
        out_ref[:, nh:] = (y_r / scale).astype(jnp.float8_e4m3fn).astype(
            jnp.float32) * scale

    return pl.pallas_call(
        body,
        out_shape=jax.ShapeDtypeStruct((m_per, n), jnp.float32),
        in_specs=[
            pl.BlockSpec(memory_space=pltpu.VMEM),
            pl.BlockSpec(memory_space=pltpu.VMEM),
        ],
        out_specs=pl.BlockSpec(memory_space=pltpu.VMEM),
        scratch_shapes=[
            pltpu.VMEM((N_SLOT, m_per, nh), jnp.float32),
            pltpu.VMEM((N_SLOT, m_per, nh), jnp.float32),
            pltpu.VMEM((8, 128), jnp.float32),
            pltpu.VMEM((N_DEV, 8, 128), jnp.float32),
            pltpu.SemaphoreType.DMA((N_SLOT, 2)),
            pltpu.SemaphoreType.DMA((N_SLOT, 2)),
            pltpu.SemaphoreType.DMA((N_SLOT, 2)),
            pltpu.SemaphoreType.DMA((N_SLOT, 2)),
            pltpu.SemaphoreType.DMA((N_DEV,)),
            pltpu.SemaphoreType.DMA((N_DEV,)),
            pltpu.SemaphoreType.REGULAR,
            pltpu.SemaphoreType.REGULAR,
        ],
        compiler_params=pltpu.CompilerParams(collective_id=0),
    )(x, w_mat)
